# baseline (speedup 1.0000x reference)
import jax
import jax.numpy as jnp
from jax import lax
from jax.experimental import pallas as pl

N, S, K = 4096, 200, 100
B_BLK = 32


def _body(in_ref, out_ref):
    ids = in_ref[...]
    iota = lax.broadcasted_iota(jnp.int32, (B_BLK * S, K), 1)
    oh = (ids == iota).astype(jnp.float32)
    out_ref[...] = oh.reshape(B_BLK, S, K)


def kernel(inputs):
    flat = inputs.reshape(N * S, 1)
    return pl.pallas_call(
        _body,
        grid=(N // B_BLK,),
        in_specs=[pl.BlockSpec((B_BLK * S, 1), lambda i: (i, 0))],
        out_specs=pl.BlockSpec((B_BLK, S, K), lambda i: (i, 0, 0)),
        out_shape=jax.ShapeDtypeStruct((N, S, K), jnp.float32),
    )(flat)


# speedup vs baseline: 1.6307x; 1.6307x over previous
import jax
import jax.numpy as jnp
from jax import lax
from jax.experimental import pallas as pl
from jax.experimental.pallas import tpu as pltpu

N, S, K = 4096, 200, 100
B_CH = 64
GRID = N // B_CH
IN_LANES = 64
ROWS = B_CH * S // IN_LANES
TOT_ROWS = N * S // IN_LANES
NBUF = 6


def _body(in_ref, out_hbm, buf, sems):
    i = pl.program_id(0)
    slot = lax.rem(i, NBUF)

    @pl.when(i >= NBUF)
    def _wait_prev():
        old = i - NBUF
        pltpu.make_async_copy(
            buf.at[slot],
            out_hbm.at[pl.ds(old * B_CH, B_CH)],
            sems.at[slot],
        ).wait()

    ids = in_ref[pl.ds(i * B_CH, B_CH), :]
    iota = lax.broadcasted_iota(jnp.int32, (B_CH, S, K), 2)
    oh = (ids[:, :, None] == iota).astype(jnp.float32)
    buf[pl.ds(slot, 1)] = oh.reshape(1, B_CH, S, K)

    pltpu.make_async_copy(
        buf.at[slot],
        out_hbm.at[pl.ds(i * B_CH, B_CH)],
        sems.at[slot],
    ).start()

    @pl.when(i == GRID - 1)
    def _drain():
        for j in range(NBUF):
            pltpu.make_async_copy(
                buf.at[j],
                out_hbm.at[pl.ds(0, B_CH)],
                sems.at[j],
            ).wait()


def kernel(inputs):
    return pl.pallas_call(
        _body,
        grid=(GRID,),
        in_specs=[pl.BlockSpec((N, S), lambda i: (0, 0))],
        out_specs=pl.BlockSpec(memory_space=pl.ANY),
        out_shape=jax.ShapeDtypeStruct((N, S, K), jnp.float32),
        scratch_shapes=[
            pltpu.VMEM((NBUF, B_CH, S, K), jnp.float32),
            pltpu.SemaphoreType.DMA((NBUF,)),
        ],
    )(inputs)
